# Initial kernel scaffold; baseline (speedup 1.0000x reference)
import jax
import jax.numpy as jnp
from jax.experimental import pallas as pl


def kernel(anchor_feat, pos_feat, neg_feat, W, b):
    raise NotImplementedError("write your pallas kernel here")


# speedup vs baseline: 1.7371x; 1.7371x over previous
import jax
import jax.numpy as jnp
from jax.experimental import pallas as pl
from jax.experimental.pallas import tpu as pltpu

_N = 50000
_D = 512
_BLOCK = 2000


def _bilinear_kernel(a_ref, p_ref, n_ref, wt_ref, b_ref, pos_out, neg_out):
    t = jnp.dot(a_ref[:], wt_ref[:], preferred_element_type=jnp.float32,
                precision=jax.lax.Precision.HIGHEST)
    bias = b_ref[0]
    logit_p = jnp.sum(p_ref[:] * t, axis=1, keepdims=True) + bias
    logit_n = jnp.sum(n_ref[:] * t, axis=1, keepdims=True) + bias
    pos_out[:] = jax.nn.sigmoid(logit_p)
    neg_out[:] = jax.nn.sigmoid(logit_n)


def kernel(anchor_feat, pos_feat, neg_feat, W, b):
    n = anchor_feat.shape[0]
    w_t = W[0].T

    grid = (n // _BLOCK,)
    feat_spec = pl.BlockSpec((_BLOCK, _D), lambda i: (i, 0))
    w_spec = pl.BlockSpec((_D, _D), lambda i: (0, 0))
    b_spec = pl.BlockSpec(memory_space=pltpu.SMEM)
    out_spec = pl.BlockSpec((_BLOCK, 1), lambda i: (i, 0))

    sc_pos, sc_neg = pl.pallas_call(
        _bilinear_kernel,
        grid=grid,
        in_specs=[feat_spec, feat_spec, feat_spec, w_spec, b_spec],
        out_specs=[out_spec, out_spec],
        out_shape=[
            jax.ShapeDtypeStruct((n, 1), jnp.float32),
            jax.ShapeDtypeStruct((n, 1), jnp.float32),
        ],
        compiler_params=pltpu.CompilerParams(
            dimension_semantics=("arbitrary",),
        ),
    )(anchor_feat, pos_feat, neg_feat, w_t, b)

    return (sc_pos[:, 0], sc_neg[:, 0])


# speedup vs baseline: 2.7977x; 1.6106x over previous
import jax
import jax.numpy as jnp
from jax.experimental import pallas as pl
from jax.experimental.pallas import tpu as pltpu

_N = 50000
_D = 512
_BLOCK = 2000


def _bilinear_kernel(a_ref, p_ref, n_ref, wt_ref, b_ref, pos_out, neg_out):
    t = jnp.dot(a_ref[:], wt_ref[:], preferred_element_type=jnp.float32)
    bias = b_ref[0]
    logit_p = jnp.sum(p_ref[:] * t, axis=1, keepdims=True) + bias
    logit_n = jnp.sum(n_ref[:] * t, axis=1, keepdims=True) + bias
    pos_out[:] = jax.nn.sigmoid(logit_p)
    neg_out[:] = jax.nn.sigmoid(logit_n)


def kernel(anchor_feat, pos_feat, neg_feat, W, b):
    n = anchor_feat.shape[0]
    w_t = W[0].T

    grid = (n // _BLOCK,)
    feat_spec = pl.BlockSpec((_BLOCK, _D), lambda i: (i, 0))
    w_spec = pl.BlockSpec((_D, _D), lambda i: (0, 0))
    b_spec = pl.BlockSpec(memory_space=pltpu.SMEM)
    out_spec = pl.BlockSpec((_BLOCK, 1), lambda i: (i, 0))

    sc_pos, sc_neg = pl.pallas_call(
        _bilinear_kernel,
        grid=grid,
        in_specs=[feat_spec, feat_spec, feat_spec, w_spec, b_spec],
        out_specs=[out_spec, out_spec],
        out_shape=[
            jax.ShapeDtypeStruct((n, 1), jnp.float32),
            jax.ShapeDtypeStruct((n, 1), jnp.float32),
        ],
        compiler_params=pltpu.CompilerParams(
            dimension_semantics=("arbitrary",),
        ),
    )(anchor_feat, pos_feat, neg_feat, w_t, b)

    return (sc_pos[:, 0], sc_neg[:, 0])


# speedup vs baseline: 2.8025x; 1.0017x over previous
import jax
import jax.numpy as jnp
from jax.experimental import pallas as pl
from jax.experimental.pallas import tpu as pltpu

_N = 50000
_D = 512
_BLOCK = 2000


def _bilinear_kernel(a_ref, p_ref, n_ref, wt_ref, b_ref, pos_out, neg_out):
    t = jnp.dot(a_ref[:], wt_ref[:], preferred_element_type=jnp.float32)
    bias = b_ref[0]
    logit_p = jnp.sum(p_ref[:] * t, axis=1, keepdims=True) + bias
    logit_n = jnp.sum(n_ref[:] * t, axis=1, keepdims=True) + bias
    pos_out[:] = jax.nn.sigmoid(logit_p)
    neg_out[:] = jax.nn.sigmoid(logit_n)


def kernel(anchor_feat, pos_feat, neg_feat, W, b):
    n = anchor_feat.shape[0]
    w_t = W[0].T

    grid = (n // _BLOCK,)
    feat_spec = pl.BlockSpec((_BLOCK, _D), lambda i: (i, 0))
    w_spec = pl.BlockSpec((_D, _D), lambda i: (0, 0))
    b_spec = pl.BlockSpec(memory_space=pltpu.SMEM)
    out_spec = pl.BlockSpec((_BLOCK, 1), lambda i: (i, 0))

    sc_pos, sc_neg = pl.pallas_call(
        _bilinear_kernel,
        grid=grid,
        in_specs=[feat_spec, feat_spec, feat_spec, w_spec, b_spec],
        out_specs=[out_spec, out_spec],
        out_shape=[
            jax.ShapeDtypeStruct((n, 1), jnp.float32),
            jax.ShapeDtypeStruct((n, 1), jnp.float32),
        ],
        compiler_params=pltpu.CompilerParams(
            dimension_semantics=("arbitrary",),
            vmem_limit_bytes=128 * 1024 * 1024,
        ),
    )(anchor_feat, pos_feat, neg_feat, w_t, b)

    return (sc_pos[:, 0], sc_neg[:, 0])
